# Initial kernel scaffold; baseline (speedup 1.0000x reference)
import jax
import jax.numpy as jnp
from jax.experimental import pallas as pl


def kernel(x, edge_index, first_edge, candidate_edges, t, gin_W1_0, gin_b1_0, gin_W2_0, gin_b2_0, gin_W1_1, gin_b1_1, gin_W2_1, gin_b2_1, gin_W1_2, gin_b1_2, gin_W2_2, gin_b2_2, t_embed_w, edge_W1, edge_b1, edge_W2, edge_b2):
    raise NotImplementedError("write your pallas kernel here")


# speedup vs baseline: 5.2966x; 5.2966x over previous
import functools

import jax
import jax.numpy as jnp
from jax import lax
from jax.experimental import pallas as pl
from jax.experimental.pallas import tpu as pltpu
from jax.experimental.pallas import tpu_sc as plsc

N = 10000
E = 320000
D = 128
H = 128
C = 4096

_NC = 2
_NS = 16
_NW = _NC * _NS

_CHUNK = 125
_NCHUNK = (E // _NW) // _CHUNK
_NPAD = 10240
_ROWS_PER_TILE = _NPAD // _NS
_ZCH = _ROWS_PER_TILE // 128


@functools.lru_cache(maxsize=None)
def _build_sc_segment_sum():
    mesh = plsc.VectorSubcoreMesh(core_axis_name="c", subcore_axis_name="s")

    @functools.partial(
        pl.kernel,
        out_type=jax.ShapeDtypeStruct((_NC, _NPAD, H), jnp.float32),
        mesh=mesh,
        scratch_types=[
            pltpu.VMEM((_NCHUNK, _CHUNK), jnp.int32),
            pltpu.VMEM((_NCHUNK, _CHUNK), jnp.int32),
            pltpu.VMEM((128, H), jnp.float32),
            pltpu.VMEM_SHARED((_NPAD, H), jnp.float32),
            pltpu.SemaphoreType.DMA,
        ],
    )
    def seg(h_hbm, src_hbm, dst_hbm, zero_hbm, out_hbm,
            src_v, dst_v, buf_v, acc_sh, sem):
        c = lax.axis_index("c")
        s = lax.axis_index("s")
        wid = s * _NC + c

        pltpu.sync_copy(zero_hbm, buf_v)
        base_row = s * _ROWS_PER_TILE
        for k in range(_ZCH):
            pltpu.sync_copy(buf_v,
                            acc_sh.at[pl.ds(base_row + k * 128, 128)])
        plsc.subcore_barrier()

        pltpu.sync_copy(src_hbm.at[wid], src_v)
        pltpu.sync_copy(dst_hbm.at[wid], dst_v)

        rows = buf_v.at[pl.ds(0, _CHUNK)]

        def body(j, carry):
            pltpu.async_copy(h_hbm.at[src_v.at[j]], rows, sem).wait()
            pltpu.sync_copy(rows, acc_sh.at[dst_v.at[j]], add=True)
            return carry

        lax.fori_loop(0, _NCHUNK, body, 0)
        plsc.subcore_barrier()

        for k in range(_ZCH):
            r0 = base_row + k * 128
            pltpu.sync_copy(acc_sh.at[pl.ds(r0, 128)], buf_v)
            pltpu.sync_copy(buf_v, out_hbm.at[c, pl.ds(r0, 128)])

    return seg


def _sc_segment_sum(h, src, dst, zero_blk):
    return _build_sc_segment_sum()(h, src, dst, zero_blk)


_GIDX = 384
_GTOT = _GIDX * _NW


@functools.lru_cache(maxsize=None)
def _build_sc_gather():
    mesh = plsc.VectorSubcoreMesh(core_axis_name="c", subcore_axis_name="s")

    @functools.partial(
        pl.kernel,
        out_type=jax.ShapeDtypeStruct((_GTOT, H), jnp.float32),
        mesh=mesh,
        scratch_types=[
            pltpu.VMEM((3, 128), jnp.int32),
            pltpu.VMEM((128, H), jnp.float32),
            pltpu.SemaphoreType.DMA,
        ],
    )
    def gat(h_hbm, idx_hbm, out_hbm, idx_v, rows_v, sem):
        c = lax.axis_index("c")
        s = lax.axis_index("s")
        wid = s * _NC + c
        pltpu.sync_copy(idx_hbm.at[wid], idx_v)
        for j in range(3):
            pltpu.async_copy(h_hbm.at[idx_v.at[j]], rows_v, sem).wait()
            pltpu.sync_copy(rows_v,
                            out_hbm.at[pl.ds(wid * _GIDX + j * 128, 128)])

    return gat


def _sc_gather(h, idx):
    return _build_sc_gather()(h, idx)


def _gin_mlp_body(h_ref, a0_ref, a1_ref, w1_ref, b1_ref, w2_ref, b2_ref,
                  o_ref):
    z = h_ref[...] + a0_ref[...] + a1_ref[...]
    m = jnp.dot(z, w1_ref[...], preferred_element_type=jnp.float32)
    m = jnp.maximum(m + b1_ref[...], 0.0)
    o_ref[...] = (jnp.dot(m, w2_ref[...], preferred_element_type=jnp.float32)
                  + b2_ref[...])


_MLP_BLK = 1000


def _tc_gin_mlp(h, a0, a1, w1, b1, w2, b2):
    grid = (N // _MLP_BLK,)
    row_spec = pl.BlockSpec((_MLP_BLK, H), lambda i: (i, 0))
    full = pl.BlockSpec((H, H), lambda i: (0, 0))
    vec = pl.BlockSpec((1, H), lambda i: (0, 0))
    return pl.pallas_call(
        _gin_mlp_body,
        grid=grid,
        in_specs=[row_spec, row_spec, row_spec, full, vec, full, vec],
        out_specs=row_spec,
        out_shape=jax.ShapeDtypeStruct((N, H), jnp.float32),
    )(h, a0, a1, w1, b1.reshape(1, H), w2, b2.reshape(1, H))


def _score_body(t_ref, xu_ref, xv_ref, fu_ref, fv_ref, temb_ref,
                w1a_ref, w1b_ref, w1c_ref, w1d_ref, w1e_ref, b1_ref,
                w2t_ref, b2_ref, o_ref):
    tv = t_ref[0]
    temb = temb_ref[pl.ds(tv, 1), :]
    fu = fu_ref[...]
    fv = fv_ref[...]
    dot = lambda a, b: jnp.dot(a, b, preferred_element_type=jnp.float32)
    base = (dot(fu + fv, w1a_ref[...])
            + dot(jnp.abs(fu - fv), w1b_ref[...])
            + dot(temb, w1e_ref[...])
            + b1_ref[...])
    xu = xu_ref[...]
    xv = xv_ref[...]
    m = dot(xu + xv, w1c_ref[...]) + dot(jnp.abs(xu - xv), w1d_ref[...])
    m = jnp.maximum(m + base, 0.0)
    o_ref[...] = jnp.sum(m * w2t_ref[...], axis=1, keepdims=True) + b2_ref[...]


def _tc_score(t, xu, xv, fu, fv, t_embed_w, edge_W1, edge_b1, edge_W2,
              edge_b2):
    tarr = jnp.asarray(t, jnp.int32).reshape(1)
    w1a = edge_W1[0:H]
    w1b = edge_W1[H:2 * H]
    w1c = edge_W1[2 * H:3 * H]
    w1d = edge_W1[3 * H:4 * H]
    w1e = edge_W1[4 * H:5 * H]
    out = pl.pallas_call(
        _score_body,
        in_specs=[
            pl.BlockSpec(memory_space=pltpu.SMEM),
            pl.BlockSpec((C, H), lambda: (0, 0)),
            pl.BlockSpec((C, H), lambda: (0, 0)),
            pl.BlockSpec((1, H), lambda: (0, 0)),
            pl.BlockSpec((1, H), lambda: (0, 0)),
            pl.BlockSpec(((1000 + 1), H), lambda: (0, 0)),
            pl.BlockSpec((H, H), lambda: (0, 0)),
            pl.BlockSpec((H, H), lambda: (0, 0)),
            pl.BlockSpec((H, H), lambda: (0, 0)),
            pl.BlockSpec((H, H), lambda: (0, 0)),
            pl.BlockSpec((H, H), lambda: (0, 0)),
            pl.BlockSpec((1, H), lambda: (0, 0)),
            pl.BlockSpec((1, H), lambda: (0, 0)),
            pl.BlockSpec((1, 1), lambda: (0, 0)),
        ],
        out_specs=pl.BlockSpec((C, 1), lambda: (0, 0)),
        out_shape=jax.ShapeDtypeStruct((C, 1), jnp.float32),
    )(tarr, xu, xv, fu, fv, t_embed_w, w1a, w1b, w1c, w1d, w1e,
      edge_b1.reshape(1, H), edge_W2.reshape(1, H), edge_b2.reshape(1, 1))
    return out.reshape(-1)


def kernel(x, edge_index, first_edge, candidate_edges, t,
           gin_W1_0, gin_b1_0, gin_W2_0, gin_b2_0,
           gin_W1_1, gin_b1_1, gin_W2_1, gin_b2_1,
           gin_W1_2, gin_b1_2, gin_W2_2, gin_b2_2,
           t_embed_w, edge_W1, edge_b1, edge_W2, edge_b2):
    src = edge_index[0].reshape(_NW, _NCHUNK, _CHUNK)
    dst = edge_index[1].reshape(_NW, _NCHUNK, _CHUNK)
    zero_blk = jnp.zeros((128, H), jnp.float32)

    layers = [(gin_W1_0, gin_b1_0, gin_W2_0, gin_b2_0),
              (gin_W1_1, gin_b1_1, gin_W2_1, gin_b2_1),
              (gin_W1_2, gin_b1_2, gin_W2_2, gin_b2_2)]
    h = x
    for w1, b1, w2, b2 in layers:
        agg = _sc_segment_sum(h, src, dst, zero_blk)
        h = _tc_gin_mlp(h, agg[0], agg[1], w1, b1, w2, b2)

    idx_all = jnp.concatenate([
        candidate_edges[:, 0], candidate_edges[:, 1], first_edge,
        jnp.zeros((_GTOT - 2 * C - 2,), jnp.int32),
    ]).reshape(_NW, 3, 128)
    g = _sc_gather(h, idx_all)
    xu = g[0:C]
    xv = g[C:2 * C]
    fu = g[2 * C:2 * C + 1]
    fv = g[2 * C + 1:2 * C + 2]
    return _tc_score(t, xu, xv, fu, fv, t_embed_w, edge_W1, edge_b1,
                     edge_W2, edge_b2)
